# baseline (speedup 1.0000x reference)
import functools

import jax
import jax.numpy as jnp
from jax import lax
from jax.experimental import pallas as pl
from jax.experimental.pallas import tpu as pltpu
from jax.experimental.pallas import tpu_sc as plsc

N_PAD = 10240
CHUNK = 80
LANES = 16
N_SUBCORES = 16
N_CORES = 2
ROWS_PER_TILE = N_PAD // N_SUBCORES
FILL_CHUNK = 80
MM_BLOCK = 1000
BST = 8


def _spmm_pipeline(h_hbm, acc, src2d, dst2d, w2d, base, srcbuf, dstbuf,
                   wbuf, bufs, gsems, ssems, esem, nch, ncols):
    nblk = nch // BST
    nbuf = len(bufs)

    def stage(b, pb, copy):
        sl = pl.ds(base + b * BST, BST)
        copy(src2d.at[sl], srcbuf.at[pb])
        copy(dst2d.at[sl], dstbuf.at[pb])
        copy(w2d.at[sl], wbuf.at[pb])

    def start(pb, t, buf, sem):
        pltpu.async_copy(h_hbm.at[srcbuf.at[pb, t]], buf, sem)

    def finish(pb, t, buf, sem):
        pltpu.make_async_copy(h_hbm.at[srcbuf.at[pb, t]], buf, sem).wait()

    def scat_start(pb, t, buf, sem):
        pltpu.async_copy(buf, acc.at[dstbuf.at[pb, t]], sem, add=True)

    def scat_wait(pb, t, buf, sem):
        pltpu.make_async_copy(buf, acc.at[dstbuf.at[pb, t]], sem).wait()

    def scale(pb, t, buf):
        def edge_group(g, _):
            bs = g * LANES
            wvec = wbuf[pb, t, pl.ds(g * LANES, LANES)]
            for i in range(LANES):
                wv = wvec[i]
                for k in range(ncols // LANES):
                    sl = pl.ds(k * LANES, LANES)
                    buf[bs + i, sl] = buf[bs + i, sl] * wv
            return 0

        lax.fori_loop(0, CHUNK // LANES, edge_group, 0)

    def prologue():
        stage(0, 0, pltpu.sync_copy)
        for t in range(nbuf - 2):
            start(0, t, bufs[t], gsems[t])

    def block(b, _):
        pb = b % 2
        pb_next = (b + 1) % 2
        has_next = b < nblk - 1

        @pl.when(has_next)
        def _():
            stage(b + 1, pb_next, lambda s, d: pltpu.async_copy(s, d, esem))

        for t in range(BST):
            bi = t % nbuf
            pi = (t + nbuf - 2) % nbuf
            finish(pb, t, bufs[bi], gsems[bi])
            scale(pb, t, bufs[bi])
            if t <= 1:
                @pl.when(b > 0)
                def _():
                    scat_wait(pb, t, bufs[pi], ssems[pi])
            else:
                scat_wait(pb, t, bufs[pi], ssems[pi])
            scat_start(pb, t, bufs[bi], ssems[bi])
            nt = t + nbuf - 2
            if nt < BST:
                start(pb, nt, bufs[nt % nbuf], gsems[nt % nbuf])
            else:
                if nt == BST:
                    @pl.when(has_next)
                    def _():
                        stage(b + 1, pb_next,
                              lambda s, d: pltpu.make_async_copy(s, d, esem).wait())

                @pl.when(has_next)
                def _():
                    start(pb_next, nt - BST, bufs[nt % nbuf], gsems[nt % nbuf])
        return 0

    def main():
        lax.fori_loop(0, nblk, block, 0)
        scat_wait(0, BST - 2, bufs[(BST - 2) % nbuf], ssems[(BST - 2) % nbuf])
        scat_wait(0, BST - 1, bufs[(BST - 1) % nbuf], ssems[(BST - 1) % nbuf])

    return prologue, main


def _make_spmm(n, nch, ncols, edge_split):
    mesh = plsc.VectorSubcoreMesh(core_axis_name="c", subcore_axis_name="s")

    @functools.partial(
        pl.kernel,
        mesh=mesh,
        out_type=[
            jax.ShapeDtypeStruct((n, ncols), jnp.float32),
            jax.ShapeDtypeStruct((n, ncols), jnp.float32),
        ],
        scratch_types=[
            pltpu.VMEM((2, BST, CHUNK), jnp.int32),
            pltpu.VMEM((2, BST, CHUNK), jnp.int32),
            pltpu.VMEM((2, BST, CHUNK), jnp.float32),
            pltpu.VMEM((CHUNK, ncols), jnp.float32),
            pltpu.VMEM((CHUNK, ncols), jnp.float32),
            pltpu.VMEM((CHUNK, ncols), jnp.float32),
            pltpu.VMEM((CHUNK, ncols), jnp.float32),
            pltpu.VMEM((2, ncols), jnp.float32),
            pltpu.VMEM_SHARED((n, ncols), jnp.float32),
        ] + [pltpu.SemaphoreType.DMA] * 9,
    )
    def spmm(h0, h1, src2d, dst2d, w2d, bias2d, s0, s1,
             srcbuf, dstbuf, wbuf, r0, r1, r2, r3, bbuf, acc, *sems):
        c = lax.axis_index("c")
        s = lax.axis_index("s")
        row0 = s * ROWS_PER_TILE

        cnt = nch // 2 if edge_split else nch
        base = s * nch + c * cnt if edge_split else s * nch

        pipes = [
            _spmm_pipeline(h_hbm, acc, src2d, dst2d, w2d, base,
                           srcbuf, dstbuf, wbuf, [r0, r1, r2, r3],
                           sems[0:4], sems[4:8], sems[8], cnt, ncols)
            for h_hbm in (h0, h1)
        ]

        for cidx, (pre, _main) in enumerate(pipes):
            @pl.when(c == cidx)
            def _(pre=pre):
                pre()

        pltpu.sync_copy(bias2d, bbuf)
        bvs = [bbuf[c, pl.ds(k * LANES, LANES)] for k in range(ncols // LANES)]

        def fill(i, _):
            for k in range(ncols // LANES):
                r3[i, pl.ds(k * LANES, LANES)] = bvs[k]
            return 0

        lax.fori_loop(0, FILL_CHUNK, fill, 0)
        for j in range(ROWS_PER_TILE // FILL_CHUNK):
            pltpu.sync_copy(r3, acc.at[pl.ds(row0 + j * FILL_CHUNK, FILL_CHUNK)])
        plsc.subcore_barrier()

        for cidx, (_pre, main) in enumerate(pipes):
            @pl.when(c == cidx)
            def _(main=main):
                main()

        plsc.subcore_barrier()

        for cidx, out_hbm in enumerate([s0, s1]):
            @pl.when(c == cidx)
            def _():
                pltpu.sync_copy(acc.at[pl.ds(row0, ROWS_PER_TILE)],
                                out_hbm.at[pl.ds(row0, ROWS_PER_TILE)])

    return spmm


def _mm1_body(x_ref, w_ref, o0_ref, o1_ref):
    xb = x_ref[...]
    o0_ref[...] = jnp.dot(xb, w_ref[:, :128], preferred_element_type=jnp.float32)
    o1_ref[...] = jnp.dot(xb, w_ref[:, 128:], preferred_element_type=jnp.float32)


def _mm2_body(s0_ref, s1_ref, w2_ref, o_ref):
    a0 = jnp.maximum(s0_ref[...], 0.0)
    a1 = jnp.maximum(s1_ref[...], 0.0)
    acc = jnp.dot(a0, w2_ref[:128, :], preferred_element_type=jnp.float32)
    acc += jnp.dot(a1, w2_ref[128:, :], preferred_element_type=jnp.float32)
    o_ref[...] = acc


def _combine_body(p0_ref, p1_ref, o_ref):
    o_ref[...] = p0_ref[...] + p1_ref[...]


def _pad_edges(src, dst, w, n_parts, e_total, n, npad):
    blk = CHUNK * BST
    per = -(-e_total // (n_parts * blk)) * blk
    e_pad = per * n_parts
    pad = e_pad - e_total
    pidx = jnp.arange(pad, dtype=jnp.int32)
    src_p = jnp.concatenate([src, pidx % n]).reshape(e_pad // CHUNK, CHUNK)
    dst_p = jnp.concatenate([dst, n + pidx % (npad - n)]).reshape(e_pad // CHUNK, CHUNK)
    w_p = jnp.concatenate([w, jnp.zeros((pad,), w.dtype)]).reshape(e_pad // CHUNK, CHUNK)
    return src_p, dst_p, w_p, per // CHUNK


def kernel(x, edge_index, edge_weight, W1, b1, W2, b2):
    n, d_in = x.shape
    npad = N_PAD
    e = edge_weight.shape[0]
    d_h = W1.shape[1]
    d_out = W2.shape[1]
    half = d_h // 2

    dst = edge_index[0].astype(jnp.int32)
    src = edge_index[1].astype(jnp.int32)

    src1, dst1, w1, nch = _pad_edges(src, dst, edge_weight, N_SUBCORES,
                                     e, n, npad)

    grid = (n // MM_BLOCK,)
    h0, h1 = pl.pallas_call(
        _mm1_body,
        grid=grid,
        in_specs=[
            pl.BlockSpec((MM_BLOCK, d_in), lambda i: (i, 0)),
            pl.BlockSpec((d_in, d_h), lambda i: (0, 0)),
        ],
        out_specs=[
            pl.BlockSpec((MM_BLOCK, half), lambda i: (i, 0)),
            pl.BlockSpec((MM_BLOCK, half), lambda i: (i, 0)),
        ],
        out_shape=[
            jax.ShapeDtypeStruct((n, half), jnp.float32),
            jax.ShapeDtypeStruct((n, half), jnp.float32),
        ],
    )(x, W1)

    s0, s1 = _make_spmm(npad, nch, half, False)(h0, h1, src1, dst1, w1,
                                                b1.reshape(2, half))

    h2 = pl.pallas_call(
        _mm2_body,
        grid=grid,
        in_specs=[
            pl.BlockSpec((MM_BLOCK, half), lambda i: (i, 0)),
            pl.BlockSpec((MM_BLOCK, half), lambda i: (i, 0)),
            pl.BlockSpec((d_h, d_out), lambda i: (0, 0)),
        ],
        out_specs=pl.BlockSpec((MM_BLOCK, d_out), lambda i: (i, 0)),
        out_shape=jax.ShapeDtypeStruct((n, d_out), jnp.float32),
    )(s0, s1, W2)

    bias2 = jnp.stack([b2, jnp.zeros_like(b2)])
    p0, p1 = _make_spmm(npad, nch, d_out, True)(h2, h2, src1, dst1, w1,
                                                bias2)

    out = pl.pallas_call(
        _combine_body,
        grid=grid,
        in_specs=[
            pl.BlockSpec((MM_BLOCK, d_out), lambda i: (i, 0)),
            pl.BlockSpec((MM_BLOCK, d_out), lambda i: (i, 0)),
        ],
        out_specs=pl.BlockSpec((MM_BLOCK, d_out), lambda i: (i, 0)),
        out_shape=jax.ShapeDtypeStruct((n, d_out), jnp.float32),
    )(p0, p1)

    return out


# speedup vs baseline: 1.0929x; 1.0929x over previous
import functools

import jax
import jax.numpy as jnp
from jax import lax
from jax.experimental import pallas as pl
from jax.experimental.pallas import tpu as pltpu
from jax.experimental.pallas import tpu_sc as plsc

N_PAD = 10240
CHUNK = 80
LANES = 16
N_SUBCORES = 16
N_CORES = 2
ROWS_PER_TILE = N_PAD // N_SUBCORES
FILL_CHUNK = 80
MM_BLOCK = 1000
BST = 8


def _spmm_pipeline(h_hbm, acc, src2d, dst2d, w2d, base, srcbuf, dstbuf,
                   wbuf, bufs, gsems, ssems, esem, nch, ncols):
    nblk = nch // BST
    nbuf = len(bufs)

    def stage(b, pb, copy):
        sl = pl.ds(base + b * BST, BST)
        copy(src2d.at[sl], srcbuf.at[pb])
        copy(dst2d.at[sl], dstbuf.at[pb])
        copy(w2d.at[sl], wbuf.at[pb])

    def start(pb, t, buf, sem):
        pltpu.async_copy(h_hbm.at[srcbuf.at[pb, t]], buf, sem)

    def finish(pb, t, buf, sem):
        pltpu.make_async_copy(h_hbm.at[srcbuf.at[pb, t]], buf, sem).wait()

    def scat_start(pb, t, buf, sem):
        pltpu.async_copy(buf, acc.at[dstbuf.at[pb, t]], sem, add=True)

    def scat_wait(pb, t, buf, sem):
        pltpu.make_async_copy(buf, acc.at[dstbuf.at[pb, t]], sem).wait()

    def scale(pb, t, buf):
        def edge_group(g, _):
            bs = g * LANES
            wvec = wbuf[pb, t, pl.ds(g * LANES, LANES)]
            for i in range(LANES):
                wv = wvec[i]
                for k in range(ncols // LANES):
                    sl = pl.ds(k * LANES, LANES)
                    buf[bs + i, sl] = buf[bs + i, sl] * wv
            return 0

        lax.fori_loop(0, CHUNK // LANES, edge_group, 0)

    def prologue():
        stage(0, 0, pltpu.sync_copy)
        for t in range(nbuf - 1):
            start(0, t, bufs[t], gsems[t])

    def block(b, _):
        pb = b % 2
        pb_next = (b + 1) % 2
        has_next = b < nblk - 1

        @pl.when(has_next)
        def _():
            stage(b + 1, pb_next, lambda s, d: pltpu.async_copy(s, d, esem))

        for t in range(BST):
            bi = t % nbuf
            pi = (t + nbuf - 1) % nbuf
            finish(pb, t, bufs[bi], gsems[bi])
            scale(pb, t, bufs[bi])
            if t == 0:
                @pl.when(b > 0)
                def _():
                    scat_wait(pb, t, bufs[pi], ssems[pi])
            else:
                scat_wait(pb, t, bufs[pi], ssems[pi])
            scat_start(pb, t, bufs[bi], ssems[bi])
            nt = t + nbuf - 1
            if nt < BST:
                start(pb, nt, bufs[nt % nbuf], gsems[nt % nbuf])
            else:
                if nt == BST:
                    @pl.when(has_next)
                    def _():
                        stage(b + 1, pb_next,
                              lambda s, d: pltpu.make_async_copy(s, d, esem).wait())

                @pl.when(has_next)
                def _():
                    start(pb_next, nt - BST, bufs[nt % nbuf], gsems[nt % nbuf])
        return 0

    def main():
        lax.fori_loop(0, nblk, block, 0)
        scat_wait(0, BST - 1, bufs[(BST - 1) % nbuf], ssems[(BST - 1) % nbuf])

    return prologue, main


def _make_spmm(n, nch, ncols, edge_split):
    mesh = plsc.VectorSubcoreMesh(core_axis_name="c", subcore_axis_name="s")

    @functools.partial(
        pl.kernel,
        mesh=mesh,
        out_type=[
            jax.ShapeDtypeStruct((n, ncols), jnp.float32),
            jax.ShapeDtypeStruct((n, ncols), jnp.float32),
        ],
        scratch_types=[
            pltpu.VMEM((2, BST, CHUNK), jnp.int32),
            pltpu.VMEM((2, BST, CHUNK), jnp.int32),
            pltpu.VMEM((2, BST, CHUNK), jnp.float32),
            pltpu.VMEM((CHUNK, ncols), jnp.float32),
            pltpu.VMEM((CHUNK, ncols), jnp.float32),
            pltpu.VMEM((CHUNK, ncols), jnp.float32),
            pltpu.VMEM((CHUNK, ncols), jnp.float32),
            pltpu.VMEM((2, ncols), jnp.float32),
            pltpu.VMEM_SHARED((n, ncols), jnp.float32),
        ] + [pltpu.SemaphoreType.DMA] * 9,
    )
    def spmm(h0, h1, src2d, dst2d, w2d, bias2d, s0, s1,
             srcbuf, dstbuf, wbuf, r0, r1, r2, r3, bbuf, acc, *sems):
        c = lax.axis_index("c")
        s = lax.axis_index("s")
        row0 = s * ROWS_PER_TILE

        cnt = nch // 2 if edge_split else nch
        base = s * nch + c * cnt if edge_split else s * nch

        pipes = [
            _spmm_pipeline(h_hbm, acc, src2d, dst2d, w2d, base,
                           srcbuf, dstbuf, wbuf, [r0, r1, r2, r3],
                           sems[0:4], sems[4:8], sems[8], cnt, ncols)
            for h_hbm in (h0, h1)
        ]

        for cidx, (pre, _main) in enumerate(pipes):
            @pl.when(c == cidx)
            def _(pre=pre):
                pre()

        pltpu.sync_copy(bias2d, bbuf)
        bvs = [bbuf[c, pl.ds(k * LANES, LANES)] for k in range(ncols // LANES)]

        def fill(i, _):
            for k in range(ncols // LANES):
                r3[i, pl.ds(k * LANES, LANES)] = bvs[k]
            return 0

        lax.fori_loop(0, FILL_CHUNK, fill, 0)
        for j in range(ROWS_PER_TILE // FILL_CHUNK):
            pltpu.sync_copy(r3, acc.at[pl.ds(row0 + j * FILL_CHUNK, FILL_CHUNK)])
        plsc.subcore_barrier()

        for cidx, (_pre, main) in enumerate(pipes):
            @pl.when(c == cidx)
            def _(main=main):
                main()

        plsc.subcore_barrier()

        for cidx, out_hbm in enumerate([s0, s1]):
            @pl.when(c == cidx)
            def _():
                pltpu.sync_copy(acc.at[pl.ds(row0, ROWS_PER_TILE)],
                                out_hbm.at[pl.ds(row0, ROWS_PER_TILE)])

    return spmm


def _mm1_body(x_ref, w_ref, o0_ref, o1_ref):
    xb = x_ref[...]
    o0_ref[...] = jnp.dot(xb, w_ref[:, :128], preferred_element_type=jnp.float32)
    o1_ref[...] = jnp.dot(xb, w_ref[:, 128:], preferred_element_type=jnp.float32)


def _mm2_body(s0_ref, s1_ref, w2_ref, o_ref):
    a0 = jnp.maximum(s0_ref[...], 0.0)
    a1 = jnp.maximum(s1_ref[...], 0.0)
    acc = jnp.dot(a0, w2_ref[:128, :], preferred_element_type=jnp.float32)
    acc += jnp.dot(a1, w2_ref[128:, :], preferred_element_type=jnp.float32)
    o_ref[...] = acc


def _combine_body(p0_ref, p1_ref, o_ref):
    o_ref[...] = p0_ref[...] + p1_ref[...]


def _pad_edges(src, dst, w, n_parts, e_total, n, npad):
    blk = CHUNK * BST
    per = -(-e_total // (n_parts * blk)) * blk
    e_pad = per * n_parts
    pad = e_pad - e_total
    pidx = jnp.arange(pad, dtype=jnp.int32)
    src_p = jnp.concatenate([src, pidx % n]).reshape(e_pad // CHUNK, CHUNK)
    dst_p = jnp.concatenate([dst, n + pidx % (npad - n)]).reshape(e_pad // CHUNK, CHUNK)
    w_p = jnp.concatenate([w, jnp.zeros((pad,), w.dtype)]).reshape(e_pad // CHUNK, CHUNK)
    return src_p, dst_p, w_p, per // CHUNK


def kernel(x, edge_index, edge_weight, W1, b1, W2, b2):
    n, d_in = x.shape
    npad = N_PAD
    e = edge_weight.shape[0]
    d_h = W1.shape[1]
    d_out = W2.shape[1]
    half = d_h // 2

    dst = edge_index[0].astype(jnp.int32)
    src = edge_index[1].astype(jnp.int32)

    src1, dst1, w1, nch = _pad_edges(src, dst, edge_weight, N_SUBCORES,
                                     e, n, npad)

    grid = (n // MM_BLOCK,)
    h0, h1 = pl.pallas_call(
        _mm1_body,
        grid=grid,
        in_specs=[
            pl.BlockSpec((MM_BLOCK, d_in), lambda i: (i, 0)),
            pl.BlockSpec((d_in, d_h), lambda i: (0, 0)),
        ],
        out_specs=[
            pl.BlockSpec((MM_BLOCK, half), lambda i: (i, 0)),
            pl.BlockSpec((MM_BLOCK, half), lambda i: (i, 0)),
        ],
        out_shape=[
            jax.ShapeDtypeStruct((n, half), jnp.float32),
            jax.ShapeDtypeStruct((n, half), jnp.float32),
        ],
    )(x, W1)

    s0, s1 = _make_spmm(npad, nch, half, False)(h0, h1, src1, dst1, w1,
                                                b1.reshape(2, half))

    h2 = pl.pallas_call(
        _mm2_body,
        grid=grid,
        in_specs=[
            pl.BlockSpec((MM_BLOCK, half), lambda i: (i, 0)),
            pl.BlockSpec((MM_BLOCK, half), lambda i: (i, 0)),
            pl.BlockSpec((d_h, d_out), lambda i: (0, 0)),
        ],
        out_specs=pl.BlockSpec((MM_BLOCK, d_out), lambda i: (i, 0)),
        out_shape=jax.ShapeDtypeStruct((n, d_out), jnp.float32),
    )(s0, s1, W2)

    bias2 = jnp.stack([b2, jnp.zeros_like(b2)])
    p0, p1 = _make_spmm(npad, nch, d_out, True)(h2, h2, src1, dst1, w1,
                                                bias2)

    out = pl.pallas_call(
        _combine_body,
        grid=grid,
        in_specs=[
            pl.BlockSpec((MM_BLOCK, d_out), lambda i: (i, 0)),
            pl.BlockSpec((MM_BLOCK, d_out), lambda i: (i, 0)),
        ],
        out_specs=pl.BlockSpec((MM_BLOCK, d_out), lambda i: (i, 0)),
        out_shape=jax.ShapeDtypeStruct((n, d_out), jnp.float32),
    )(p0, p1)

    return out


# speedup vs baseline: 1.1280x; 1.0321x over previous
import functools

import jax
import jax.numpy as jnp
from jax import lax
from jax.experimental import pallas as pl
from jax.experimental.pallas import tpu as pltpu
from jax.experimental.pallas import tpu_sc as plsc

N_PAD = 10240
CHUNK = 80
LANES = 16
N_SUBCORES = 16
N_CORES = 2
ROWS_PER_TILE = N_PAD // N_SUBCORES
FILL_CHUNK = 80
MM_BLOCK = 2000
BST = 8


def _spmm_pipeline(h_hbm, acc, src2d, dst2d, w2d, base, srcbuf, dstbuf,
                   wbuf, bufs, gsems, ssems, esem, nch, ncols):
    nblk = nch // BST
    nbuf = len(bufs)

    def stage(b, pb, copy):
        sl = pl.ds(base + b * BST, BST)
        copy(src2d.at[sl], srcbuf.at[pb])
        copy(dst2d.at[sl], dstbuf.at[pb])
        copy(w2d.at[sl], wbuf.at[pb])

    def start(pb, t, buf, sem):
        pltpu.async_copy(h_hbm.at[srcbuf.at[pb, t]], buf, sem)

    def finish(pb, t, buf, sem):
        pltpu.make_async_copy(h_hbm.at[srcbuf.at[pb, t]], buf, sem).wait()

    def scat_start(pb, t, buf, sem):
        pltpu.async_copy(buf, acc.at[dstbuf.at[pb, t]], sem, add=True)

    def scat_wait(pb, t, buf, sem):
        pltpu.make_async_copy(buf, acc.at[dstbuf.at[pb, t]], sem).wait()

    def scale(pb, t, buf):
        def edge_group(g, _):
            bs = g * LANES
            wvec = wbuf[pb, t, pl.ds(g * LANES, LANES)]
            for i in range(LANES):
                wv = wvec[i]
                for k in range(ncols // LANES):
                    sl = pl.ds(k * LANES, LANES)
                    buf[bs + i, sl] = buf[bs + i, sl] * wv
            return 0

        lax.fori_loop(0, CHUNK // LANES, edge_group, 0)

    def prologue():
        stage(0, 0, pltpu.sync_copy)
        for t in range(nbuf - 1):
            start(0, t, bufs[t], gsems[t])

    def block(b, _):
        pb = b % 2
        pb_next = (b + 1) % 2
        has_next = b < nblk - 1

        @pl.when(has_next)
        def _():
            stage(b + 1, pb_next, lambda s, d: pltpu.async_copy(s, d, esem))

        for t in range(BST):
            bi = t % nbuf
            pi = (t + nbuf - 1) % nbuf
            finish(pb, t, bufs[bi], gsems[bi])
            scale(pb, t, bufs[bi])
            if t == 0:
                @pl.when(b > 0)
                def _():
                    scat_wait(pb, t, bufs[pi], ssems[pi])
            else:
                scat_wait(pb, t, bufs[pi], ssems[pi])
            scat_start(pb, t, bufs[bi], ssems[bi])
            nt = t + nbuf - 1
            if nt < BST:
                start(pb, nt, bufs[nt % nbuf], gsems[nt % nbuf])
            else:
                if nt == BST:
                    @pl.when(has_next)
                    def _():
                        stage(b + 1, pb_next,
                              lambda s, d: pltpu.make_async_copy(s, d, esem).wait())

                @pl.when(has_next)
                def _():
                    start(pb_next, nt - BST, bufs[nt % nbuf], gsems[nt % nbuf])
        return 0

    def main():
        lax.fori_loop(0, nblk, block, 0)
        scat_wait(0, BST - 1, bufs[(BST - 1) % nbuf], ssems[(BST - 1) % nbuf])

    return prologue, main


def _make_spmm(n, nch, ncols, edge_split):
    mesh = plsc.VectorSubcoreMesh(core_axis_name="c", subcore_axis_name="s")

    @functools.partial(
        pl.kernel,
        mesh=mesh,
        out_type=[
            jax.ShapeDtypeStruct((n, ncols), jnp.float32),
            jax.ShapeDtypeStruct((n, ncols), jnp.float32),
        ],
        scratch_types=[
            pltpu.VMEM((2, BST, CHUNK), jnp.int32),
            pltpu.VMEM((2, BST, CHUNK), jnp.int32),
            pltpu.VMEM((2, BST, CHUNK), jnp.float32),
            pltpu.VMEM((CHUNK, ncols), jnp.float32),
            pltpu.VMEM((CHUNK, ncols), jnp.float32),
            pltpu.VMEM((CHUNK, ncols), jnp.float32),
            pltpu.VMEM((CHUNK, ncols), jnp.float32),
            pltpu.VMEM((2, ncols), jnp.float32),
            pltpu.VMEM_SHARED((n, ncols), jnp.float32),
        ] + [pltpu.SemaphoreType.DMA] * 9,
    )
    def spmm(h0, h1, src2d, dst2d, w2d, bias2d, s0, s1,
             srcbuf, dstbuf, wbuf, r0, r1, r2, r3, bbuf, acc, *sems):
        c = lax.axis_index("c")
        s = lax.axis_index("s")
        row0 = s * ROWS_PER_TILE

        cnt = nch // 2 if edge_split else nch
        base = s * nch + c * cnt if edge_split else s * nch

        pipes = [
            _spmm_pipeline(h_hbm, acc, src2d, dst2d, w2d, base,
                           srcbuf, dstbuf, wbuf, [r0, r1, r2, r3],
                           sems[0:4], sems[4:8], sems[8], cnt, ncols)
            for h_hbm in (h0, h1)
        ]

        for cidx, (pre, _main) in enumerate(pipes):
            @pl.when(c == cidx)
            def _(pre=pre):
                pre()

        pltpu.sync_copy(bias2d, bbuf)
        bvs = [bbuf[c, pl.ds(k * LANES, LANES)] for k in range(ncols // LANES)]

        def fill(i, _):
            for k in range(ncols // LANES):
                r3[i, pl.ds(k * LANES, LANES)] = bvs[k]
            return 0

        lax.fori_loop(0, FILL_CHUNK, fill, 0)
        for j in range(ROWS_PER_TILE // FILL_CHUNK):
            pltpu.sync_copy(r3, acc.at[pl.ds(row0 + j * FILL_CHUNK, FILL_CHUNK)])
        plsc.subcore_barrier()

        for cidx, (_pre, main) in enumerate(pipes):
            @pl.when(c == cidx)
            def _(main=main):
                main()

        plsc.subcore_barrier()

        for cidx, out_hbm in enumerate([s0, s1]):
            @pl.when(c == cidx)
            def _():
                pltpu.sync_copy(acc.at[pl.ds(row0, ROWS_PER_TILE)],
                                out_hbm.at[pl.ds(row0, ROWS_PER_TILE)])

    return spmm


def _mm1_body(x_ref, w_ref, o0_ref, o1_ref):
    xb = x_ref[...]
    o0_ref[...] = jnp.dot(xb, w_ref[:, :128], preferred_element_type=jnp.float32)
    o1_ref[...] = jnp.dot(xb, w_ref[:, 128:], preferred_element_type=jnp.float32)


def _mm2_body(s0_ref, s1_ref, w2_ref, o_ref):
    a0 = jnp.maximum(s0_ref[...], 0.0)
    a1 = jnp.maximum(s1_ref[...], 0.0)
    acc = jnp.dot(a0, w2_ref[:128, :], preferred_element_type=jnp.float32)
    acc += jnp.dot(a1, w2_ref[128:, :], preferred_element_type=jnp.float32)
    o_ref[...] = acc


def _combine_body(p0_ref, p1_ref, o_ref):
    o_ref[...] = p0_ref[...] + p1_ref[...]


def _pad_edges(src, dst, w, n_parts, e_total, n, npad):
    blk = CHUNK * BST
    per = -(-e_total // (n_parts * blk)) * blk
    e_pad = per * n_parts
    pad = e_pad - e_total
    pidx = jnp.arange(pad, dtype=jnp.int32)
    src_p = jnp.concatenate([src, pidx % n]).reshape(e_pad // CHUNK, CHUNK)
    dst_p = jnp.concatenate([dst, n + pidx % (npad - n)]).reshape(e_pad // CHUNK, CHUNK)
    w_p = jnp.concatenate([w, jnp.zeros((pad,), w.dtype)]).reshape(e_pad // CHUNK, CHUNK)
    return src_p, dst_p, w_p, per // CHUNK


def kernel(x, edge_index, edge_weight, W1, b1, W2, b2):
    n, d_in = x.shape
    npad = N_PAD
    e = edge_weight.shape[0]
    d_h = W1.shape[1]
    d_out = W2.shape[1]
    half = d_h // 2

    dst = edge_index[0].astype(jnp.int32)
    src = edge_index[1].astype(jnp.int32)

    src1, dst1, w1, nch = _pad_edges(src, dst, edge_weight, N_SUBCORES,
                                     e, n, npad)

    grid = (n // MM_BLOCK,)
    h0, h1 = pl.pallas_call(
        _mm1_body,
        grid=grid,
        in_specs=[
            pl.BlockSpec((MM_BLOCK, d_in), lambda i: (i, 0)),
            pl.BlockSpec((d_in, d_h), lambda i: (0, 0)),
        ],
        out_specs=[
            pl.BlockSpec((MM_BLOCK, half), lambda i: (i, 0)),
            pl.BlockSpec((MM_BLOCK, half), lambda i: (i, 0)),
        ],
        out_shape=[
            jax.ShapeDtypeStruct((n, half), jnp.float32),
            jax.ShapeDtypeStruct((n, half), jnp.float32),
        ],
    )(x, W1)

    s0, s1 = _make_spmm(npad, nch, half, False)(h0, h1, src1, dst1, w1,
                                                b1.reshape(2, half))

    h2 = pl.pallas_call(
        _mm2_body,
        grid=grid,
        in_specs=[
            pl.BlockSpec((MM_BLOCK, half), lambda i: (i, 0)),
            pl.BlockSpec((MM_BLOCK, half), lambda i: (i, 0)),
            pl.BlockSpec((d_h, d_out), lambda i: (0, 0)),
        ],
        out_specs=pl.BlockSpec((MM_BLOCK, d_out), lambda i: (i, 0)),
        out_shape=jax.ShapeDtypeStruct((n, d_out), jnp.float32),
    )(s0, s1, W2)

    bias2 = jnp.stack([b2, jnp.zeros_like(b2)])
    p0, p1 = _make_spmm(npad, nch, d_out, True)(h2, h2, src1, dst1, w1,
                                                bias2)

    out = pl.pallas_call(
        _combine_body,
        grid=grid,
        in_specs=[
            pl.BlockSpec((MM_BLOCK, d_out), lambda i: (i, 0)),
            pl.BlockSpec((MM_BLOCK, d_out), lambda i: (i, 0)),
        ],
        out_specs=pl.BlockSpec((MM_BLOCK, d_out), lambda i: (i, 0)),
        out_shape=jax.ShapeDtypeStruct((n, d_out), jnp.float32),
    )(p0, p1)

    return out


# speedup vs baseline: 1.1524x; 1.0217x over previous
import functools

import jax
import jax.numpy as jnp
from jax import lax
from jax.experimental import pallas as pl
from jax.experimental.pallas import tpu as pltpu
from jax.experimental.pallas import tpu_sc as plsc

N_PAD = 10240
CHUNK = 80
LANES = 16
N_SUBCORES = 16
N_CORES = 2
ROWS_PER_TILE = N_PAD // N_SUBCORES
FILL_CHUNK = 80
MM_BLOCK = 5000
BST = 8


def _spmm_pipeline(h_hbm, acc, src2d, dst2d, w2d, base, srcbuf, dstbuf,
                   wbuf, bufs, gsems, ssems, esem, nch, ncols):
    nblk = nch // BST
    nbuf = len(bufs)

    def stage(b, pb, copy):
        sl = pl.ds(base + b * BST, BST)
        copy(src2d.at[sl], srcbuf.at[pb])
        copy(dst2d.at[sl], dstbuf.at[pb])
        copy(w2d.at[sl], wbuf.at[pb])

    def start(pb, t, buf, sem):
        pltpu.async_copy(h_hbm.at[srcbuf.at[pb, t]], buf, sem)

    def finish(pb, t, buf, sem):
        pltpu.make_async_copy(h_hbm.at[srcbuf.at[pb, t]], buf, sem).wait()

    def scat_start(pb, t, buf, sem):
        pltpu.async_copy(buf, acc.at[dstbuf.at[pb, t]], sem, add=True)

    def scat_wait(pb, t, buf, sem):
        pltpu.make_async_copy(buf, acc.at[dstbuf.at[pb, t]], sem).wait()

    def scale(pb, t, buf):
        def edge_group(g, _):
            bs = g * LANES
            wvec = wbuf[pb, t, pl.ds(g * LANES, LANES)]
            for i in range(LANES):
                wv = wvec[i]
                for k in range(ncols // LANES):
                    sl = pl.ds(k * LANES, LANES)
                    buf[bs + i, sl] = buf[bs + i, sl] * wv
            return 0

        lax.fori_loop(0, CHUNK // LANES, edge_group, 0)

    def prologue():
        stage(0, 0, pltpu.sync_copy)
        for t in range(nbuf - 1):
            start(0, t, bufs[t], gsems[t])

    def block(b, _):
        pb = b % 2
        pb_next = (b + 1) % 2
        has_next = b < nblk - 1

        @pl.when(has_next)
        def _():
            stage(b + 1, pb_next, lambda s, d: pltpu.async_copy(s, d, esem))

        for t in range(BST):
            bi = t % nbuf
            pi = (t + nbuf - 1) % nbuf
            finish(pb, t, bufs[bi], gsems[bi])
            scale(pb, t, bufs[bi])
            if t == 0:
                @pl.when(b > 0)
                def _():
                    scat_wait(pb, t, bufs[pi], ssems[pi])
            else:
                scat_wait(pb, t, bufs[pi], ssems[pi])
            scat_start(pb, t, bufs[bi], ssems[bi])
            nt = t + nbuf - 1
            if nt < BST:
                start(pb, nt, bufs[nt % nbuf], gsems[nt % nbuf])
            else:
                if nt == BST:
                    @pl.when(has_next)
                    def _():
                        stage(b + 1, pb_next,
                              lambda s, d: pltpu.make_async_copy(s, d, esem).wait())

                @pl.when(has_next)
                def _():
                    start(pb_next, nt - BST, bufs[nt % nbuf], gsems[nt % nbuf])
        return 0

    def main():
        lax.fori_loop(0, nblk, block, 0)
        scat_wait(0, BST - 1, bufs[(BST - 1) % nbuf], ssems[(BST - 1) % nbuf])

    return prologue, main


def _make_spmm(n, nch, ncols, edge_split):
    mesh = plsc.VectorSubcoreMesh(core_axis_name="c", subcore_axis_name="s")

    @functools.partial(
        pl.kernel,
        mesh=mesh,
        out_type=[
            jax.ShapeDtypeStruct((n, ncols), jnp.float32),
            jax.ShapeDtypeStruct((n, ncols), jnp.float32),
        ],
        scratch_types=[
            pltpu.VMEM((2, BST, CHUNK), jnp.int32),
            pltpu.VMEM((2, BST, CHUNK), jnp.int32),
            pltpu.VMEM((2, BST, CHUNK), jnp.float32),
            pltpu.VMEM((CHUNK, ncols), jnp.float32),
            pltpu.VMEM((CHUNK, ncols), jnp.float32),
            pltpu.VMEM((CHUNK, ncols), jnp.float32),
            pltpu.VMEM((CHUNK, ncols), jnp.float32),
            pltpu.VMEM((2, ncols), jnp.float32),
            pltpu.VMEM_SHARED((n, ncols), jnp.float32),
        ] + [pltpu.SemaphoreType.DMA] * 9,
    )
    def spmm(h0, h1, src2d, dst2d, w2d, bias2d, s0, s1,
             srcbuf, dstbuf, wbuf, r0, r1, r2, r3, bbuf, acc, *sems):
        c = lax.axis_index("c")
        s = lax.axis_index("s")
        row0 = s * ROWS_PER_TILE

        cnt = nch // 2 if edge_split else nch
        base = s * nch + c * cnt if edge_split else s * nch

        pipes = [
            _spmm_pipeline(h_hbm, acc, src2d, dst2d, w2d, base,
                           srcbuf, dstbuf, wbuf, [r0, r1, r2, r3],
                           sems[0:4], sems[4:8], sems[8], cnt, ncols)
            for h_hbm in (h0, h1)
        ]

        for cidx, (pre, _main) in enumerate(pipes):
            @pl.when(c == cidx)
            def _(pre=pre):
                pre()

        pltpu.sync_copy(bias2d, bbuf)
        bvs = [bbuf[c, pl.ds(k * LANES, LANES)] for k in range(ncols // LANES)]

        def fill(i, _):
            for k in range(ncols // LANES):
                r3[i, pl.ds(k * LANES, LANES)] = bvs[k]
            return 0

        lax.fori_loop(0, FILL_CHUNK, fill, 0)
        for j in range(ROWS_PER_TILE // FILL_CHUNK):
            pltpu.sync_copy(r3, acc.at[pl.ds(row0 + j * FILL_CHUNK, FILL_CHUNK)])
        plsc.subcore_barrier()

        for cidx, (_pre, main) in enumerate(pipes):
            @pl.when(c == cidx)
            def _(main=main):
                main()

        plsc.subcore_barrier()

        for cidx, out_hbm in enumerate([s0, s1]):
            @pl.when(c == cidx)
            def _():
                pltpu.sync_copy(acc.at[pl.ds(row0, ROWS_PER_TILE)],
                                out_hbm.at[pl.ds(row0, ROWS_PER_TILE)])

    return spmm


def _mm1_body(x_ref, w_ref, o0_ref, o1_ref):
    xb = x_ref[...]
    o0_ref[...] = jnp.dot(xb, w_ref[:, :128], preferred_element_type=jnp.float32)
    o1_ref[...] = jnp.dot(xb, w_ref[:, 128:], preferred_element_type=jnp.float32)


def _mm2_body(s0_ref, s1_ref, w2_ref, o_ref):
    a0 = jnp.maximum(s0_ref[...], 0.0)
    a1 = jnp.maximum(s1_ref[...], 0.0)
    acc = jnp.dot(a0, w2_ref[:128, :], preferred_element_type=jnp.float32)
    acc += jnp.dot(a1, w2_ref[128:, :], preferred_element_type=jnp.float32)
    o_ref[...] = acc


def _combine_body(p0_ref, p1_ref, o_ref):
    o_ref[...] = p0_ref[...] + p1_ref[...]


def _pad_edges(src, dst, w, n_parts, e_total, n, npad):
    blk = CHUNK * BST
    per = -(-e_total // (n_parts * blk)) * blk
    e_pad = per * n_parts
    pad = e_pad - e_total
    pidx = jnp.arange(pad, dtype=jnp.int32)
    src_p = jnp.concatenate([src, pidx % n]).reshape(e_pad // CHUNK, CHUNK)
    dst_p = jnp.concatenate([dst, n + pidx % (npad - n)]).reshape(e_pad // CHUNK, CHUNK)
    w_p = jnp.concatenate([w, jnp.zeros((pad,), w.dtype)]).reshape(e_pad // CHUNK, CHUNK)
    return src_p, dst_p, w_p, per // CHUNK


def kernel(x, edge_index, edge_weight, W1, b1, W2, b2):
    n, d_in = x.shape
    npad = N_PAD
    e = edge_weight.shape[0]
    d_h = W1.shape[1]
    d_out = W2.shape[1]
    half = d_h // 2

    dst = edge_index[0].astype(jnp.int32)
    src = edge_index[1].astype(jnp.int32)

    src1, dst1, w1, nch = _pad_edges(src, dst, edge_weight, N_SUBCORES,
                                     e, n, npad)

    grid = (n // MM_BLOCK,)
    h0, h1 = pl.pallas_call(
        _mm1_body,
        grid=grid,
        in_specs=[
            pl.BlockSpec((MM_BLOCK, d_in), lambda i: (i, 0)),
            pl.BlockSpec((d_in, d_h), lambda i: (0, 0)),
        ],
        out_specs=[
            pl.BlockSpec((MM_BLOCK, half), lambda i: (i, 0)),
            pl.BlockSpec((MM_BLOCK, half), lambda i: (i, 0)),
        ],
        out_shape=[
            jax.ShapeDtypeStruct((n, half), jnp.float32),
            jax.ShapeDtypeStruct((n, half), jnp.float32),
        ],
    )(x, W1)

    s0, s1 = _make_spmm(npad, nch, half, False)(h0, h1, src1, dst1, w1,
                                                b1.reshape(2, half))

    h2 = pl.pallas_call(
        _mm2_body,
        grid=grid,
        in_specs=[
            pl.BlockSpec((MM_BLOCK, half), lambda i: (i, 0)),
            pl.BlockSpec((MM_BLOCK, half), lambda i: (i, 0)),
            pl.BlockSpec((d_h, d_out), lambda i: (0, 0)),
        ],
        out_specs=pl.BlockSpec((MM_BLOCK, d_out), lambda i: (i, 0)),
        out_shape=jax.ShapeDtypeStruct((n, d_out), jnp.float32),
    )(s0, s1, W2)

    bias2 = jnp.stack([b2, jnp.zeros_like(b2)])
    p0, p1 = _make_spmm(npad, nch, d_out, True)(h2, h2, src1, dst1, w1,
                                                bias2)

    out = pl.pallas_call(
        _combine_body,
        grid=grid,
        in_specs=[
            pl.BlockSpec((MM_BLOCK, d_out), lambda i: (i, 0)),
            pl.BlockSpec((MM_BLOCK, d_out), lambda i: (i, 0)),
        ],
        out_specs=pl.BlockSpec((MM_BLOCK, d_out), lambda i: (i, 0)),
        out_shape=jax.ShapeDtypeStruct((n, d_out), jnp.float32),
    )(p0, p1)

    return out
